# Initial kernel scaffold; baseline (speedup 1.0000x reference)
import jax
import jax.numpy as jnp
from jax.experimental import pallas as pl


def kernel(H, G, K, user, item, feat_idx, segment_ids):
    raise NotImplementedError("write your pallas kernel here")


# speedup vs baseline: 6.0785x; 6.0785x over previous
import functools

import jax
import jax.numpy as jnp
from jax import lax
from jax.experimental import pallas as pl
from jax.experimental.pallas import tpu as pltpu
from jax.experimental.pallas import tpu_sc as plsc

B = 16384
F = 512
D = 128
T = 819200
NC = 2
NS = 16
NW = NC * NS
RC = 32
NCH = B // RC
CPW = NCH // NW
TB = 2048
SENT = B * F

_mesh = plsc.VectorSubcoreMesh(
    core_axis_name="c", subcore_axis_name="s", num_cores=NC, num_subcores=NS
)
_sc_params = pltpu.CompilerParams(needs_layout_passes=False)


@functools.partial(
    pl.kernel,
    out_type=jax.ShapeDtypeStruct((B, D), jnp.float32),
    mesh=_mesh,
    compiler_params=_sc_params,
    scratch_types=[
        pltpu.VMEM((2, 128), jnp.int32),
        pltpu.VMEM((2, 128, D), jnp.float32),
        pltpu.SemaphoreType.DMA((2,)),
    ],
)
def _gather_h(user_hbm, h_tab_hbm, out_hbm, idx_v, rows_v, sem_g):
    w = lax.axis_index("s") * NC + lax.axis_index("c")
    nq = (B // NW) // 128

    def start(q, slot):
        base = w * (B // NW) + q * 128
        pltpu.sync_copy(user_hbm.at[pl.ds(base, 128)], idx_v.at[slot])
        pltpu.async_copy(h_tab_hbm.at[idx_v.at[slot]], rows_v.at[slot],
                         sem_g.at[slot])

    start(0, 0)
    for q in range(nq):
        slot = q % 2
        if q + 1 < nq:
            start(q + 1, 1 - slot)
        base = w * (B // NW) + q * 128
        pltpu.make_async_copy(h_tab_hbm.at[idx_v.at[slot]], rows_v.at[slot],
                              sem_g.at[slot]).wait()
        pltpu.sync_copy(rows_v.at[slot], out_hbm.at[pl.ds(base, 128), :])


def _mm_body(h_ref, g_ref, o_ref):
    o_ref[...] = lax.dot_general(
        h_ref[...], g_ref[...],
        (((1,), (1,)), ((), ())),
        preferred_element_type=jnp.float32,
    )


@functools.partial(
    pl.kernel,
    out_type=jax.ShapeDtypeStruct((NW, 16), jnp.float32),
    mesh=_mesh,
    compiler_params=_sc_params,
    scratch_types=[
        pltpu.VMEM((32,), jnp.int32),
        pltpu.VMEM((2, RC), jnp.int32),
        pltpu.VMEM((2, RC, F), jnp.float32),
        pltpu.VMEM((2, RC, F), jnp.float32),
        pltpu.VMEM((TB,), jnp.int32),
        pltpu.VMEM((16,), jnp.float32),
        pltpu.SemaphoreType.DMA((2,)),
    ],
)
def _main_sc(k_tab_hbm, z_hbm, user_hbm, cidx_hbm, offs_hbm, out_hbm,
             offs_v, uidx_v, krows_v, zrows_v, cbuf_v, acc_v, sem_kz):
    w = lax.axis_index("s") * NC + lax.axis_index("c")
    pltpu.sync_copy(offs_hbm.at[pl.ds(w * CPW, 32)], offs_v)
    o_lo = offs_v[pl.ds(0, 16)]
    o_hi = offs_v[pl.ds(16, 16)]

    def off(j):
        return o_lo[j] if j < 16 else o_hi[j - 16]

    def start_chunk(j, slot):
        r0 = (w * CPW + j) * RC
        pltpu.sync_copy(user_hbm.at[pl.ds(r0, RC)], uidx_v.at[slot])
        pltpu.async_copy(k_tab_hbm.at[uidx_v.at[slot]], krows_v.at[slot],
                         sem_kz.at[slot])
        pltpu.async_copy(z_hbm.at[pl.ds(r0, RC), :], zrows_v.at[slot],
                         sem_kz.at[slot])

    start_chunk(0, 0)
    acc = jnp.zeros((16,), jnp.float32)
    for j in range(CPW):
        slot = j % 2
        if j + 1 < CPW:
            start_chunk(j + 1, 1 - slot)
        r0 = (w * CPW + j) * RC
        pltpu.make_async_copy(k_tab_hbm.at[uidx_v.at[slot]],
                              krows_v.at[slot], sem_kz.at[slot]).wait()
        pltpu.make_async_copy(z_hbm.at[pl.ds(r0, RC), :],
                              zrows_v.at[slot], sem_kz.at[slot]).wait()
        t0 = off(j)
        t1 = off(j + 1)
        lo = r0 * F
        hi = lo + RC * F
        t0a = t0 & (-8)
        nb = (t1 - t0a + TB - 1) // TB

        def batch_body(bi, acc):
            bs = pl.multiple_of(t0a + bi * TB, 8)
            pltpu.sync_copy(cidx_hbm.at[pl.ds(bs, TB)], cbuf_v)

            def grp(i, acc):
                cv = cbuf_v[pl.ds(i * 16, 16)]
                m = (cv >= lo) & (cv < hi)
                loc = cv - lo
                r = (loc >> 9) & (RC - 1)
                col = loc & (F - 1)
                kv = plsc.load_gather(krows_v.at[slot], [r, col], mask=m)
                zv = plsc.load_gather(zrows_v.at[slot], [r, col], mask=m)
                return acc + jnp.where(m, kv * zv, 0.0)

            return lax.fori_loop(0, TB // 16, grp, acc)

        acc = lax.fori_loop(0, nb, batch_body, acc)

    acc_v[...] = acc
    pltpu.sync_copy(acc_v, out_hbm.at[w])


def kernel(H, G, K, user, item, feat_idx, segment_ids):
    del item
    cidx = segment_ids * F + feat_idx
    cidx = jnp.concatenate([cidx, jnp.full((TB,), SENT, jnp.int32)])
    bounds = jnp.searchsorted(
        segment_ids, jnp.arange(0, B + 1, RC, dtype=jnp.int32), side="left"
    ).astype(jnp.int32)
    offs = jnp.concatenate([bounds, jnp.full((31,), T, jnp.int32)])

    h_u = _gather_h(user, H)

    BM = 1024
    z = pl.pallas_call(
        _mm_body,
        grid=(B // BM,),
        in_specs=[
            pl.BlockSpec((BM, D), lambda i: (i, 0)),
            pl.BlockSpec((F, D), lambda i: (0, 0)),
        ],
        out_specs=pl.BlockSpec((BM, F), lambda i: (i, 0)),
        out_shape=jax.ShapeDtypeStruct((B, F), jnp.float32),
    )(h_u, G)

    partials = _main_sc(K, z, user, cidx, offs)
    return jnp.sum(partials)


# speedup vs baseline: 6.5128x; 1.0714x over previous
import functools

import jax
import jax.numpy as jnp
from jax import lax
from jax.experimental import pallas as pl
from jax.experimental.pallas import tpu as pltpu
from jax.experimental.pallas import tpu_sc as plsc

B = 16384
F = 512
D = 128
T = 819200
NC = 2
NS = 16
NW = NC * NS
RC = 64
NCH = B // RC
CPW = NCH // NW
TB = 2048
NOFF = 264

_mesh = plsc.VectorSubcoreMesh(
    core_axis_name="c", subcore_axis_name="s", num_cores=NC, num_subcores=NS
)
_sc_params = pltpu.CompilerParams(needs_layout_passes=False)


@functools.partial(
    pl.kernel,
    out_type=jax.ShapeDtypeStruct((B, D), jnp.float32),
    mesh=_mesh,
    compiler_params=_sc_params,
    scratch_types=[
        pltpu.VMEM((2, 128), jnp.int32),
        pltpu.VMEM((2, 128, D), jnp.float32),
        pltpu.SemaphoreType.DMA((2,)),
    ],
)
def _gather_h(user_hbm, h_tab_hbm, out_hbm, idx_v, rows_v, sem_g):
    w = lax.axis_index("s") * NC + lax.axis_index("c")
    nq = (B // NW) // 128

    def start(q, slot):
        base = w * (B // NW) + q * 128
        pltpu.sync_copy(user_hbm.at[pl.ds(base, 128)], idx_v.at[slot])
        pltpu.async_copy(h_tab_hbm.at[idx_v.at[slot]], rows_v.at[slot],
                         sem_g.at[slot])

    start(0, 0)
    for q in range(nq):
        slot = q % 2
        if q + 1 < nq:
            start(q + 1, 1 - slot)
        base = w * (B // NW) + q * 128
        pltpu.make_async_copy(h_tab_hbm.at[idx_v.at[slot]], rows_v.at[slot],
                              sem_g.at[slot]).wait()
        pltpu.sync_copy(rows_v.at[slot], out_hbm.at[pl.ds(base, 128), :])


def _mm_body(h_ref, g_ref, o_ref):
    o_ref[...] = lax.dot_general(
        h_ref[...], g_ref[...],
        (((1,), (1,)), ((), ())),
        preferred_element_type=jnp.float32,
    )


@functools.partial(
    pl.kernel,
    out_type=jax.ShapeDtypeStruct((NW, 16), jnp.float32),
    mesh=_mesh,
    compiler_params=_sc_params,
    scratch_types=[
        pltpu.VMEM((16,), jnp.int32),
        pltpu.VMEM((RC,), jnp.int32),
        pltpu.VMEM((RC, F), jnp.float32),
        pltpu.VMEM((RC, F), jnp.float32),
        pltpu.VMEM((TB,), jnp.int32),
        pltpu.VMEM((TB,), jnp.int32),
        pltpu.VMEM((16,), jnp.float32),
        pltpu.SemaphoreType.DMA,
        pltpu.SemaphoreType.DMA,
        pltpu.SemaphoreType.DMA((2,)),
    ],
)
def _main_sc(k_tab_hbm, z_hbm, user_hbm, seg_hbm, feat_hbm, offs_hbm, out_hbm,
             offs_v, uidx_v, krows_v, zrows_v, sbuf_v, fbuf_v, acc_v,
             sem_k, sem_z, sem_i):
    w = lax.axis_index("s") * NC + lax.axis_index("c")
    pltpu.sync_copy(offs_hbm.at[pl.ds(w * CPW, 16)], offs_v)
    offv = offs_v[...]
    lanes = lax.iota(jnp.int32, 16)

    acc = jnp.zeros((16,), jnp.float32)
    for j in range(CPW):
        t0 = offv[j]
        t1 = offv[j + 1]
        r0 = (w * CPW + j) * RC
        pltpu.sync_copy(user_hbm.at[pl.ds(r0, RC)], uidx_v)
        cp_k = pltpu.async_copy(k_tab_hbm.at[uidx_v], krows_v, sem_k)
        cp_z = pltpu.async_copy(z_hbm.at[pl.ds(r0, RC), :], zrows_v, sem_z)
        cp_k.wait()
        cp_z.wait()
        lo = r0 * F
        hi = lo + RC * F
        t0a = t0 & (-8)
        nb = (t1 - t0a + TB - 1) // TB

        def batch_body(bi, acc):
            wm = t0a + bi * TB
            bs = pl.multiple_of(jnp.minimum(wm, T - TB), 8)
            ci = pltpu.async_copy(seg_hbm.at[pl.ds(bs, TB)], sbuf_v,
                                  sem_i.at[0])
            cf = pltpu.async_copy(feat_hbm.at[pl.ds(bs, TB)], fbuf_v,
                                  sem_i.at[1])
            ci.wait()
            cf.wait()

            def grp(i, acc):
                sv = sbuf_v[pl.ds(i * 16, 16)]
                fv = fbuf_v[pl.ds(i * 16, 16)]
                cv = (sv << 9) | fv
                pos = lanes + (bs + i * 16)
                m = (cv >= lo) & (cv < hi)
                m = m & (pos >= wm) & (pos < wm + TB)
                loc = cv - lo
                r = (loc >> 9) & (RC - 1)
                col = loc & (F - 1)
                kv = plsc.load_gather(krows_v, [r, col], mask=m)
                zv = plsc.load_gather(zrows_v, [r, col], mask=m)
                return acc + jnp.where(m, kv * zv, 0.0)

            return lax.fori_loop(0, TB // 16, grp, acc)

        acc = lax.fori_loop(0, nb, batch_body, acc)

    acc_v[...] = acc
    pltpu.sync_copy(acc_v, out_hbm.at[w])


def kernel(H, G, K, user, item, feat_idx, segment_ids):
    del item
    queries = jnp.minimum(
        jnp.arange(0, NOFF * RC, RC, dtype=jnp.int32), B
    )
    offs = jnp.searchsorted(segment_ids, queries, side="left").astype(jnp.int32)

    h_u = _gather_h(user, H)

    BM = 1024
    z = pl.pallas_call(
        _mm_body,
        grid=(B // BM,),
        in_specs=[
            pl.BlockSpec((BM, D), lambda i: (i, 0)),
            pl.BlockSpec((F, D), lambda i: (0, 0)),
        ],
        out_specs=pl.BlockSpec((BM, F), lambda i: (i, 0)),
        out_shape=jax.ShapeDtypeStruct((B, F), jnp.float32),
    )(h_u, G)

    partials = _main_sc(K, z, user, segment_ids, feat_idx, offs)
    return jnp.sum(partials)
